# initial kernel scaffold (unmeasured)
import jax
import jax.numpy as jnp
from jax import lax
from jax.experimental import pallas as pl
from jax.experimental.pallas import tpu as pltpu

N_DEV = 4
N_LAYERS = 3
FROM_LEFT, FROM_RIGHT, DIAG = 0, 1, 2


def kernel(x, Win0, Wout0, Win1, Wout1, Win2, Wout2):
    b, d = x.shape
    out_rows = b // N_DEV

    def body(x_ref, win0, wout0, win1, wout1, win2, wout2,
             out_ref, partial_ref, comm_ref, send_sems, recv_sems):
        my = lax.axis_index("i")
        left = lax.rem(my + N_DEV - 1, N_DEV)
        right = lax.rem(my + 1, N_DEV)

        barrier = pltpu.get_barrier_semaphore()
        for nbr in (left, right):
            pl.semaphore_signal(
                barrier, inc=1,
                device_id=(nbr,), device_id_type=pl.DeviceIdType.MESH,
            )
        pl.semaphore_wait(barrier, 2)

        wins = [win0, win1, win2]
        wouts = [wout0, wout1, wout2]

        xb = x_ref[...].astype(jnp.bfloat16)
        total = None
        for l in range(N_LAYERS):
            h = jnp.dot(xb, wins[l][...].astype(jnp.bfloat16),
                        preferred_element_type=jnp.float32)
            h = jnp.maximum(h, 0.0).astype(jnp.bfloat16)
            part = jnp.dot(h, wouts[l][...].astype(jnp.bfloat16),
                           preferred_element_type=jnp.float32)
            partial_ref[...] = part

            send_l = pltpu.make_async_remote_copy(
                src_ref=partial_ref,
                dst_ref=comm_ref.at[l, FROM_RIGHT],
                send_sem=send_sems.at[l, 0],
                recv_sem=recv_sems.at[l, FROM_RIGHT],
                device_id=(left,), device_id_type=pl.DeviceIdType.MESH,
            )
            send_r = pltpu.make_async_remote_copy(
                src_ref=partial_ref,
                dst_ref=comm_ref.at[l, FROM_LEFT],
                send_sem=send_sems.at[l, 1],
                recv_sem=recv_sems.at[l, FROM_LEFT],
                device_id=(right,), device_id_type=pl.DeviceIdType.MESH,
            )
            send_l.start()
            send_r.start()

            send_l.wait_recv()
            fwd = pltpu.make_async_remote_copy(
                src_ref=comm_ref.at[l, FROM_RIGHT],
                dst_ref=comm_ref.at[l, DIAG],
                send_sem=send_sems.at[l, 2],
                recv_sem=recv_sems.at[l, DIAG],
                device_id=(left,), device_id_type=pl.DeviceIdType.MESH,
            )
            fwd.start()

            send_r.wait_recv()
            fwd.wait_recv()

            total = (partial_ref[...] + comm_ref[l, FROM_LEFT]
                     + comm_ref[l, FROM_RIGHT] + comm_ref[l, DIAG])

            send_l.wait_send()
            send_r.wait_send()
            fwd.wait_send()

            if l < N_LAYERS - 1:
                xb = total.astype(jnp.bfloat16)

        out_ref[...] = lax.dynamic_slice_in_dim(total, my * out_rows,
                                                out_rows, axis=0)

    return pl.pallas_call(
        body,
        out_shape=jax.ShapeDtypeStruct((out_rows, d), jnp.float32),
        in_specs=[pl.BlockSpec(memory_space=pltpu.VMEM)] * 7,
        out_specs=pl.BlockSpec(memory_space=pltpu.VMEM),
        scratch_shapes=[
            pltpu.VMEM((b, d), jnp.float32),
            pltpu.VMEM((N_LAYERS, 3, b, d), jnp.float32),
            pltpu.SemaphoreType.DMA((N_LAYERS, 3)),
            pltpu.SemaphoreType.DMA((N_LAYERS, 3)),
        ],
        compiler_params=pltpu.CompilerParams(collective_id=0),
    )(x, Win0, Wout0, Win1, Wout1, Win2, Wout2)


# baseline (device time: 52960 ns/iter reference)
import jax
import jax.numpy as jnp
from jax import lax
from jax.experimental import pallas as pl
from jax.experimental.pallas import tpu as pltpu

N_DEV = 4
N_LAYERS = 3
FROM_LEFT, FROM_RIGHT, DIAG = 0, 1, 2


def kernel(x, Win0, Wout0, Win1, Wout1, Win2, Wout2):
    b, d = x.shape
    out_rows = b // N_DEV

    def body(x_ref, win0, wout0, win1, wout1, win2, wout2,
             out_ref, partial_ref, comm_ref, send_sems, recv_sems):
        my = lax.axis_index("i")
        left = lax.rem(my + N_DEV - 1, N_DEV)
        right = lax.rem(my + 1, N_DEV)

        barrier = pltpu.get_barrier_semaphore()
        for nbr in (left, right):
            pl.semaphore_signal(
                barrier, inc=1,
                device_id=(nbr,), device_id_type=pl.DeviceIdType.MESH,
            )
        pl.semaphore_wait(barrier, 2)

        wins = [win0, win1, win2]
        wouts = [wout0, wout1, wout2]

        xb = x_ref[...].astype(jnp.bfloat16)
        total = None
        for l in range(N_LAYERS):
            h = jnp.dot(xb, wins[l][...].astype(jnp.bfloat16),
                        preferred_element_type=jnp.float32)
            h = jnp.maximum(h, 0.0).astype(jnp.bfloat16)
            part = jnp.dot(h, wouts[l][...].astype(jnp.bfloat16),
                           preferred_element_type=jnp.float32)
            partial_ref[...] = part

            send_l = pltpu.make_async_remote_copy(
                src_ref=partial_ref,
                dst_ref=comm_ref.at[l, FROM_RIGHT],
                send_sem=send_sems.at[l, 0],
                recv_sem=recv_sems.at[l, FROM_RIGHT],
                device_id=(left,), device_id_type=pl.DeviceIdType.MESH,
            )
            send_r = pltpu.make_async_remote_copy(
                src_ref=partial_ref,
                dst_ref=comm_ref.at[l, FROM_LEFT],
                send_sem=send_sems.at[l, 1],
                recv_sem=recv_sems.at[l, FROM_LEFT],
                device_id=(right,), device_id_type=pl.DeviceIdType.MESH,
            )
            send_l.start()
            send_r.start()

            send_l.wait_recv()
            fwd = pltpu.make_async_remote_copy(
                src_ref=comm_ref.at[l, FROM_RIGHT],
                dst_ref=comm_ref.at[l, DIAG],
                send_sem=send_sems.at[l, 2],
                recv_sem=recv_sems.at[l, DIAG],
                device_id=(left,), device_id_type=pl.DeviceIdType.MESH,
            )
            fwd.start()

            send_r.wait_recv()
            fwd.wait_recv()

            if l < N_LAYERS - 1:
                total = (partial_ref[...] + comm_ref[l, FROM_LEFT]
                         + comm_ref[l, FROM_RIGHT] + comm_ref[l, DIAG])
                xb = total.astype(jnp.bfloat16)
            else:
                rows = pl.ds(my * out_rows, out_rows)
                out_ref[...] = (partial_ref[rows, :]
                                + comm_ref[l, FROM_LEFT, rows, :]
                                + comm_ref[l, FROM_RIGHT, rows, :]
                                + comm_ref[l, DIAG, rows, :])

            send_l.wait_send()
            send_r.wait_send()
            fwd.wait_send()

    return pl.pallas_call(
        body,
        out_shape=jax.ShapeDtypeStruct((out_rows, d), jnp.float32),
        in_specs=[pl.BlockSpec(memory_space=pltpu.VMEM)] * 7,
        out_specs=pl.BlockSpec(memory_space=pltpu.VMEM),
        scratch_shapes=[
            pltpu.VMEM((b, d), jnp.float32),
            pltpu.VMEM((N_LAYERS, 3, b, d), jnp.float32),
            pltpu.SemaphoreType.DMA((N_LAYERS, 3)),
            pltpu.SemaphoreType.DMA((N_LAYERS, 3)),
        ],
        compiler_params=pltpu.CompilerParams(
            collective_id=0,
            vmem_limit_bytes=100 * 1024 * 1024,
        ),
    )(x, Win0, Wout0, Win1, Wout1, Win2, Wout2)


# device time: 40881 ns/iter; 1.2955x vs baseline; 1.2955x over previous
import jax
import jax.numpy as jnp
from jax import lax
from jax.experimental import pallas as pl
from jax.experimental.pallas import tpu as pltpu

N_DEV = 4
N_LAYERS = 3
FROM_LEFT, FROM_RIGHT, FROM_DIAG = 0, 1, 2


def kernel(x, Win0, Wout0, Win1, Wout1, Win2, Wout2):
    b, d = x.shape
    out_rows = b // N_DEV

    def body(x_ref, win0, wout0, win1, wout1, win2, wout2,
             out_ref, partial_ref, comm_ref, send_sems, recv_sems):
        my = lax.axis_index("i")
        left = lax.rem(my + N_DEV - 1, N_DEV)
        right = lax.rem(my + 1, N_DEV)
        diag = lax.rem(my + 2, N_DEV)

        barrier = pltpu.get_barrier_semaphore()
        for nbr in (left, right, diag):
            pl.semaphore_signal(
                barrier, inc=1,
                device_id=(nbr,), device_id_type=pl.DeviceIdType.MESH,
            )
        pl.semaphore_wait(barrier, 3)

        wins = [win0, win1, win2]
        wouts = [wout0, wout1, wout2]

        xb = x_ref[...].astype(jnp.bfloat16)
        for l in range(N_LAYERS):
            h = jnp.dot(xb, wins[l][...].astype(jnp.bfloat16),
                        preferred_element_type=jnp.float32)
            h = jnp.maximum(h, 0.0).astype(jnp.bfloat16)
            part = jnp.dot(h, wouts[l][...].astype(jnp.bfloat16),
                           preferred_element_type=jnp.float32)
            partial_ref[...] = part.astype(jnp.bfloat16)

            rdmas = []
            for j, (peer, slot) in enumerate((
                (left, FROM_RIGHT), (right, FROM_LEFT), (diag, FROM_DIAG),
            )):
                r = pltpu.make_async_remote_copy(
                    src_ref=partial_ref,
                    dst_ref=comm_ref.at[l, slot],
                    send_sem=send_sems.at[l, j],
                    recv_sem=recv_sems.at[l, slot],
                    device_id=(peer,), device_id_type=pl.DeviceIdType.MESH,
                )
                r.start()
                rdmas.append(r)

            for r in rdmas:
                r.wait_recv()

            if l < N_LAYERS - 1:
                total = (part
                         + comm_ref[l, FROM_LEFT].astype(jnp.float32)
                         + comm_ref[l, FROM_RIGHT].astype(jnp.float32)
                         + comm_ref[l, FROM_DIAG].astype(jnp.float32))
                xb = total.astype(jnp.bfloat16)
            else:
                rows = pl.ds(my * out_rows, out_rows)
                out_ref[...] = (
                    partial_ref[rows, :].astype(jnp.float32)
                    + comm_ref[l, FROM_LEFT, rows, :].astype(jnp.float32)
                    + comm_ref[l, FROM_RIGHT, rows, :].astype(jnp.float32)
                    + comm_ref[l, FROM_DIAG, rows, :].astype(jnp.float32))

            for r in rdmas:
                r.wait_send()

    return pl.pallas_call(
        body,
        out_shape=jax.ShapeDtypeStruct((out_rows, d), jnp.float32),
        in_specs=[pl.BlockSpec(memory_space=pltpu.VMEM)] * 7,
        out_specs=pl.BlockSpec(memory_space=pltpu.VMEM),
        scratch_shapes=[
            pltpu.VMEM((b, d), jnp.bfloat16),
            pltpu.VMEM((N_LAYERS, 3, b, d), jnp.bfloat16),
            pltpu.SemaphoreType.DMA((N_LAYERS, 3)),
            pltpu.SemaphoreType.DMA((N_LAYERS, 3)),
        ],
        compiler_params=pltpu.CompilerParams(
            collective_id=0,
            vmem_limit_bytes=100 * 1024 * 1024,
        ),
    )(x, Win0, Wout0, Win1, Wout1, Win2, Wout2)


# device time: 31410 ns/iter; 1.6861x vs baseline; 1.3015x over previous
import jax
import jax.numpy as jnp
from jax import lax
from jax.experimental import pallas as pl
from jax.experimental.pallas import tpu as pltpu

N_DEV = 4
N_LAYERS = 3
FROM_LEFT, FROM_RIGHT, FROM_DIAG = 0, 1, 2


def kernel(x, Win0, Wout0, Win1, Wout1, Win2, Wout2):
    b, d = x.shape
    out_rows = b // N_DEV

    def body(x_ref, win0, wout0, win1, wout1, win2, wout2,
             out_ref, partial_ref, comm_ref, win_buf, wout_buf,
             send_sems, recv_sems, win_dma_sems, wout_dma_sems):
        my = lax.axis_index("i")
        left = lax.rem(my + N_DEV - 1, N_DEV)
        right = lax.rem(my + 1, N_DEV)
        diag = lax.rem(my + 2, N_DEV)

        wins = [win0, win1, win2]
        wouts = [wout0, wout1, wout2]

        def start_weight_dma(l):
            cw = pltpu.make_async_copy(
                wins[l], win_buf.at[l % 2], win_dma_sems.at[l % 2])
            co = pltpu.make_async_copy(
                wouts[l], wout_buf.at[l % 2], wout_dma_sems.at[l % 2])
            cw.start()
            co.start()
            return cw, co

        pending = start_weight_dma(0)

        barrier = pltpu.get_barrier_semaphore()
        for nbr in (left, right, diag):
            pl.semaphore_signal(
                barrier, inc=1,
                device_id=(nbr,), device_id_type=pl.DeviceIdType.MESH,
            )
        pl.semaphore_wait(barrier, 3)

        xb = x_ref[...].astype(jnp.bfloat16)
        for l in range(N_LAYERS):
            cw, co = pending
            if l + 1 < N_LAYERS:
                pending = start_weight_dma(l + 1)

            cw.wait()
            h = jnp.dot(xb, win_buf[l % 2].astype(jnp.bfloat16),
                        preferred_element_type=jnp.float32)
            h = jnp.maximum(h, 0.0).astype(jnp.bfloat16)
            co.wait()
            part = jnp.dot(h, wout_buf[l % 2].astype(jnp.bfloat16),
                           preferred_element_type=jnp.float32)
            partial_ref[...] = part.astype(jnp.bfloat16)

            rdmas = []
            for j, (peer, slot) in enumerate((
                (left, FROM_RIGHT), (right, FROM_LEFT), (diag, FROM_DIAG),
            )):
                r = pltpu.make_async_remote_copy(
                    src_ref=partial_ref,
                    dst_ref=comm_ref.at[l, slot],
                    send_sem=send_sems.at[l, j],
                    recv_sem=recv_sems.at[l, slot],
                    device_id=(peer,), device_id_type=pl.DeviceIdType.MESH,
                )
                r.start()
                rdmas.append(r)

            for r in rdmas:
                r.wait_recv()

            if l < N_LAYERS - 1:
                total = (part
                         + comm_ref[l, FROM_LEFT].astype(jnp.float32)
                         + comm_ref[l, FROM_RIGHT].astype(jnp.float32)
                         + comm_ref[l, FROM_DIAG].astype(jnp.float32))
                xb = total.astype(jnp.bfloat16)
            else:
                rows = pl.ds(my * out_rows, out_rows)
                out_ref[...] = (
                    partial_ref[rows, :].astype(jnp.float32)
                    + comm_ref[l, FROM_LEFT, rows, :].astype(jnp.float32)
                    + comm_ref[l, FROM_RIGHT, rows, :].astype(jnp.float32)
                    + comm_ref[l, FROM_DIAG, rows, :].astype(jnp.float32))

            for r in rdmas:
                r.wait_send()

    d_in, h_in = Win0.shape
    return pl.pallas_call(
        body,
        out_shape=jax.ShapeDtypeStruct((out_rows, d), jnp.float32),
        in_specs=[pl.BlockSpec(memory_space=pltpu.VMEM)]
        + [pl.BlockSpec(memory_space=pl.ANY)] * 6,
        out_specs=pl.BlockSpec(memory_space=pltpu.VMEM),
        scratch_shapes=[
            pltpu.VMEM((b, d), jnp.bfloat16),
            pltpu.VMEM((N_LAYERS, 3, b, d), jnp.bfloat16),
            pltpu.VMEM((2, d_in, h_in), jnp.float32),
            pltpu.VMEM((2, h_in, d_in), jnp.float32),
            pltpu.SemaphoreType.DMA((N_LAYERS, 3)),
            pltpu.SemaphoreType.DMA((N_LAYERS, 3)),
            pltpu.SemaphoreType.DMA((2,)),
            pltpu.SemaphoreType.DMA((2,)),
        ],
        compiler_params=pltpu.CompilerParams(
            collective_id=0,
            vmem_limit_bytes=100 * 1024 * 1024,
        ),
    )(x, Win0, Wout0, Win1, Wout1, Win2, Wout2)
